# baseline (speedup 1.0000x reference)
import jax
import jax.numpy as jnp
from jax.experimental import pallas as pl
from jax.experimental.pallas import tpu as pltpu

_BPS = 8
_F32 = jnp.float32


def _lrelu(v):
    return jnp.maximum(v, 0.01 * v)


def _dg(lhs, rhs, dims):
    return jax.lax.dot_general(lhs, rhs, (dims, ((), ())),
                               preferred_element_type=_F32)


def _gcn_body(a_ref, x_ref, k1_ref, k2_ref, b1_ref, b2_ref, o_ref):
    a = a_ref[...]
    x3 = x_ref[...]
    p1 = _dg(x3, a, ((1,), (1,)))
    h1 = _dg(p1, k1_ref[...], ((1,), (0,)))
    h1 = _lrelu(h1 + b1_ref[...])
    p2 = _dg(h1, a, ((1,), (1,)))
    h2 = _dg(p2, k2_ref[...], ((1,), (0,)))
    o_ref[...] = _lrelu(h2 + b2_ref[...])


def kernel(X, A, W1, b1, W2, b2):
    B, N, T, F_in = X.shape
    F_sp = W1.shape[1]
    assert F_in == F_sp, "flattened-column layout assumes F_in == F_sp"
    S = T * F_in

    Xr = X.reshape(B, N, S)
    eye = jnp.eye(T, dtype=X.dtype)
    K1 = jnp.kron(eye, W1)
    K2 = jnp.kron(eye, W2)
    b1t = jnp.tile(b1, T)[None, None, :]
    b2t = jnp.tile(b2, T)[None, None, :]

    out = pl.pallas_call(
        _gcn_body,
        grid=(B // _BPS,),
        in_specs=[
            pl.BlockSpec((N, N), lambda j: (0, 0)),
            pl.BlockSpec((_BPS, N, S), lambda j: (j, 0, 0)),
            pl.BlockSpec((S, S), lambda j: (0, 0)),
            pl.BlockSpec((S, S), lambda j: (0, 0)),
            pl.BlockSpec((1, 1, S), lambda j: (0, 0, 0)),
            pl.BlockSpec((1, 1, S), lambda j: (0, 0, 0)),
        ],
        out_specs=pl.BlockSpec((_BPS, N, S), lambda j: (j, 0, 0)),
        out_shape=jax.ShapeDtypeStruct((B, N, S), _F32),
        compiler_params=pltpu.CompilerParams(
            dimension_semantics=("arbitrary",),
        ),
    )(A, Xr, K1, K2, b1t, b2t)

    return out.reshape(B, N, T, F_sp)


# speedup vs baseline: 1.1961x; 1.1961x over previous
import jax
import jax.numpy as jnp
from jax.experimental import pallas as pl
from jax.experimental.pallas import tpu as pltpu

_BPS = 8
_KW = 128
_F32 = jnp.float32


def _lrelu(v):
    return jnp.maximum(v, 0.01 * v)


def _gcn_body(a_ref, x_ref, k1_ref, k2_ref, b1_ref, b2_ref, o_ref):
    a = a_ref[...]
    k1 = k1_ref[...]
    k2 = k2_ref[...]
    b1 = b1_ref[...]
    b2 = b2_ref[...]
    S = o_ref.shape[2]
    W = x_ref.shape[1]
    p1 = jnp.dot(a, x_ref[...], preferred_element_type=_F32)
    hs = []
    for c in range(W // _KW):
        h = jnp.dot(p1[:, c * _KW:(c + 1) * _KW], k1, preferred_element_type=_F32)
        hs.append(_lrelu(h + b1))
    h1 = jnp.concatenate(hs, axis=1)
    p2 = jnp.dot(a, h1, preferred_element_type=_F32)
    for i in range(_BPS):
        h = jnp.dot(p2[:, i * S:(i + 1) * S], k2, preferred_element_type=_F32)
        o_ref[i] = _lrelu(h + b2)


def kernel(X, A, W1, b1, W2, b2):
    B, N, T, F_in = X.shape
    F_sp = W1.shape[1]
    assert F_in == F_sp, "flattened-column layout assumes F_in == F_sp"
    S = T * F_in
    C = B * S

    Xmat = jnp.transpose(X, (1, 0, 2, 3)).reshape(N, C)

    nblk = _KW // F_in
    K1 = jnp.kron(jnp.eye(nblk, dtype=X.dtype), W1)
    K2 = jnp.kron(jnp.eye(T, dtype=X.dtype), W2)
    b1t = jnp.tile(b1, nblk)[None, :]
    b2t = jnp.tile(b2, T)[None, :]

    tile = _BPS * S
    out = pl.pallas_call(
        _gcn_body,
        grid=(B // _BPS,),
        in_specs=[
            pl.BlockSpec((N, N), lambda j: (0, 0)),
            pl.BlockSpec((N, tile), lambda j: (0, j)),
            pl.BlockSpec((_KW, _KW), lambda j: (0, 0)),
            pl.BlockSpec((S, S), lambda j: (0, 0)),
            pl.BlockSpec((1, _KW), lambda j: (0, 0)),
            pl.BlockSpec((1, S), lambda j: (0, 0)),
        ],
        out_specs=pl.BlockSpec((_BPS, N, S), lambda j: (j, 0, 0)),
        out_shape=jax.ShapeDtypeStruct((B, N, S), _F32),
        compiler_params=pltpu.CompilerParams(
            dimension_semantics=("arbitrary",),
        ),
    )(A, Xmat, K1, K2, b1t, b2t)

    return out.reshape(B, N, T, F_sp)


# speedup vs baseline: 1.3572x; 1.1347x over previous
import jax
import jax.numpy as jnp
from jax.experimental import pallas as pl
from jax.experimental.pallas import tpu as pltpu

_TILE = 1024
_KW = 128


def _lrelu(v):
    return jnp.maximum(v, 0.01 * v)


def _dot(lhs, rhs):
    return jnp.dot(lhs, rhs, preferred_element_type=jnp.float32)


def _gcn_body(a_ref, x_ref, k1_ref, k2_ref, b1_ref, b2_ref, o_ref):
    a = a_ref[...]
    k1 = k1_ref[...]
    k2 = k2_ref[...]
    b1 = b1_ref[...]
    b2 = b2_ref[...]
    p1 = _dot(a, x_ref[...])
    hs = []
    for c in range(_TILE // _KW):
        h = _dot(p1[:, c * _KW:(c + 1) * _KW], k1)
        hs.append(_lrelu(h + b1))
    h1 = jnp.concatenate(hs, axis=1)
    p2 = _dot(a, h1)
    for c in range(_TILE // _KW):
        h = _dot(p2[:, c * _KW:(c + 1) * _KW], k2)
        o_ref[:, c * _KW:(c + 1) * _KW] = _lrelu(h + b2)


def kernel(X, A, W1, b1, W2, b2):
    B, N, T, F_in = X.shape
    F_sp = W1.shape[1]
    assert F_in == F_sp, "flattened-column layout assumes F_in == F_sp"
    C = B * T * F_in

    Xmat = jnp.transpose(X, (1, 0, 2, 3)).reshape(N, C)

    nblk = _KW // F_in
    eye = jnp.eye(nblk, dtype=X.dtype)
    K1 = jnp.kron(eye, W1)
    K2 = jnp.kron(eye, W2)
    b1t = jnp.tile(b1, nblk)[None, :]
    b2t = jnp.tile(b2, nblk)[None, :]

    out = pl.pallas_call(
        _gcn_body,
        grid=(C // _TILE,),
        in_specs=[
            pl.BlockSpec((N, N), lambda j: (0, 0)),
            pl.BlockSpec((N, _TILE), lambda j: (0, j)),
            pl.BlockSpec((_KW, _KW), lambda j: (0, 0)),
            pl.BlockSpec((_KW, _KW), lambda j: (0, 0)),
            pl.BlockSpec((1, _KW), lambda j: (0, 0)),
            pl.BlockSpec((1, _KW), lambda j: (0, 0)),
        ],
        out_specs=pl.BlockSpec((N, _TILE), lambda j: (0, j)),
        out_shape=jax.ShapeDtypeStruct((N, C), jnp.float32),
        compiler_params=pltpu.CompilerParams(
            dimension_semantics=("arbitrary",),
        ),
    )(A, Xmat, K1, K2, b1t, b2t)

    return jnp.transpose(out.reshape(N, B, T, F_sp), (1, 0, 2, 3))
